# Initial kernel scaffold; baseline (speedup 1.0000x reference)
import jax
import jax.numpy as jnp
from jax.experimental import pallas as pl


def kernel(x):
    raise NotImplementedError("write your pallas kernel here")


# speedup vs baseline: 19.9075x; 19.9075x over previous
import jax
import jax.numpy as jnp
from jax.experimental import pallas as pl
from jax.experimental.pallas import tpu as pltpu

_P = 0.5


def _offsets(B, T):
    prob = jnp.array([_P / 2.0, 1.0 - _P, _P / 2.0], dtype=jnp.float32)
    key = jax.random.key(42)
    d = jax.random.categorical(key, jnp.log(prob), shape=(B, T)) - 1
    d = d.at[:, 0].set(jnp.clip(d[:, 0], 0, 1))
    d = d.at[:, -1].set(jnp.clip(d[:, -1], -1, 0))
    return d.astype(jnp.int32)


def _jitter_body(d_ref, x_ref, o_ref):
    xb = x_ref[0]
    d = d_ref[0]
    xm = jnp.concatenate([xb[:, :1], xb[:, :-1]], axis=1)
    xp = jnp.concatenate([xb[:, 1:], xb[:, -1:]], axis=1)
    o_ref[0] = jnp.where(d < 0, xm, jnp.where(d > 0, xp, xb))


def kernel(x):
    B, C, T = x.shape
    d = _offsets(B, T).reshape(B, 1, T)
    out = pl.pallas_call(
        _jitter_body,
        grid=(B,),
        in_specs=[
            pl.BlockSpec((1, 1, T), lambda b: (b, 0, 0)),
            pl.BlockSpec((1, C, T), lambda b: (b, 0, 0)),
        ],
        out_specs=pl.BlockSpec((1, C, T), lambda b: (b, 0, 0)),
        out_shape=jax.ShapeDtypeStruct((B, C, T), x.dtype),
    )(d, x)
    return out
